# baseline (speedup 1.0000x reference)
import functools

import jax
import jax.numpy as jnp
from jax import lax
from jax.experimental import pallas as pl
from jax.experimental.pallas import tpu as pltpu
from jax.experimental.pallas import tpu_sc as plsc

HIDDEN = 1024
ROWS = 4 * 4096
SEQ = 4096
PAIR = 1024
G0 = 29
G1 = 35
N0 = G0 * 16
N1 = G1 * 16


def _make_kernel():
    mesh = plsc.VectorSubcoreMesh(core_axis_name="c", subcore_axis_name="s")

    @functools.partial(
        pl.kernel,
        mesh=mesh,
        out_type=jax.ShapeDtypeStruct((ROWS, HIDDEN), jnp.float32),
        scratch_types=[
            pltpu.VMEM((N1,), jnp.int32),
            pltpu.VMEM((2, HIDDEN), jnp.float32),
            pltpu.SemaphoreType.DMA,
            pltpu.SemaphoreType.DMA,
            pltpu.SemaphoreType.DMA,
        ],
    )
    def body(ids_hbm, table_hbm, out_hbm, ids_v, table_v, sem, psem1, psem2):
        c = lax.axis_index("c")
        s = lax.axis_index("s")
        base = s * PAIR + c * N0
        h_tab = pltpu.async_copy(table_hbm, table_v, psem1)
        h_ids = pltpu.async_copy(ids_hbm.at[pl.ds(base, N1)], ids_v, psem2)
        h_tab.wait()
        h_ids.wait()
        n_groups = G0 + (G1 - G0) * c

        def group(g, carry):
            r0 = g * 16
            vec = ids_v[pl.ds(r0, 16)]
            for j in range(16):
                pltpu.async_copy(table_v.at[vec[j]],
                                 out_hbm.at[base + r0 + j], sem)
            return carry

        lax.fori_loop(0, n_groups, group, 0, unroll=1)
        pltpu.make_async_copy(
            out_hbm.at[pl.ds(base, N0)], out_hbm.at[pl.ds(base, N0)], sem
        ).wait()

        @pl.when(c == 1)
        def _():
            pltpu.make_async_copy(
                out_hbm.at[pl.ds(base, N1 - N0)],
                out_hbm.at[pl.ds(base, N1 - N0)],
                sem,
            ).wait()

    return body


_kernel = _make_kernel()


@jax.jit
def kernel(token_type_ids, table):
    b, s = token_type_ids.shape
    out = _kernel(token_type_ids.astype(jnp.int32).reshape(-1), table)
    return out.reshape(b, s, HIDDEN)
